# baseline (speedup 1.0000x reference)
import functools

import jax
import jax.numpy as jnp
from jax import lax
from jax.experimental import pallas as pl
from jax.experimental.pallas import tpu as pltpu
from jax.experimental.pallas import tpu_sc as plsc

_B = 4096
_D = 16
_GRP = 8
_NBUF = 3


def _gather_sc(PT, QT, user_id, item_id):
    info = plsc.get_sparse_core_info()
    nc, ns = info.num_cores, info.num_subcores
    nw = nc * ns
    b_per_w = _B // nw
    n_grp = b_per_w // _GRP

    mesh = plsc.VectorSubcoreMesh(core_axis_name="c", subcore_axis_name="s")

    @functools.partial(
        pl.kernel,
        mesh=mesh,
        out_type=[
            jax.ShapeDtypeStruct((_D, _B), jnp.float32),
            jax.ShapeDtypeStruct((_D, _B), jnp.float32),
        ],
        scratch_types=[
            pltpu.VMEM((b_per_w,), jnp.int32),
            pltpu.VMEM((b_per_w,), jnp.int32),
            pltpu.VMEM((_NBUF, _GRP, _D, 128), jnp.float32),
            pltpu.VMEM((_NBUF, _GRP, _D, 128), jnp.float32),
            pltpu.VMEM((_D, b_per_w), jnp.float32),
            pltpu.VMEM((_D, b_per_w), jnp.float32),
            pltpu.SemaphoreType.DMA,
            pltpu.SemaphoreType.DMA,
        ],
        compiler_params=pltpu.CompilerParams(needs_layout_passes=False),
    )
    def gather(pt_hbm, qt_hbm, uid_hbm, iid_hbm, put_hbm, qit_hbm,
               uidx_v, iidx_v, pwin_v, qwin_v, pcols_v, qcols_v, psem, qsem):
        wid = lax.axis_index("s") * nc + lax.axis_index("c")
        base = wid * b_per_w
        pltpu.sync_copy(uid_hbm.at[pl.ds(base, b_per_w)], uidx_v)
        pltpu.sync_copy(iid_hbm.at[pl.ds(base, b_per_w)], iidx_v)
        rows16 = jnp.arange(_D, dtype=jnp.int32)

        def extract(state):
            g, pcp, qcp, lanes_p, lanes_q = state
            bank = g % _NBUF
            for j in range(_GRP):
                col = _GRP * g + j
                colvec = jnp.full((_D,), col, dtype=jnp.int32)
                pcp[j].wait()
                pv = plsc.load_gather(
                    pwin_v.at[bank].at[j],
                    [rows16, jnp.full((_D,), lanes_p[j], dtype=jnp.int32)])
                plsc.store_scatter(pcols_v, [rows16, colvec], pv)
                qcp[j].wait()
                qv = plsc.load_gather(
                    qwin_v.at[bank].at[j],
                    [rows16, jnp.full((_D,), lanes_q[j], dtype=jnp.int32)])
                plsc.store_scatter(qcols_v, [rows16, colvec], qv)

        pending = []
        uvec = ivec = None
        for g in range(n_grp):
            if g % 2 == 0:
                uvec = uidx_v[pl.ds(16 * (g // 2), 16)]
                ivec = iidx_v[pl.ds(16 * (g // 2), 16)]
            off = _GRP * (g % 2)
            bank = g % _NBUF
            pcp, qcp, lanes_p, lanes_q = [], [], [], []
            for j in range(_GRP):
                u = uvec[off + j]
                i = ivec[off + j]
                ublk = pl.multiple_of(
                    lax.shift_right_logical(u, 7) * 128, 128)
                iblk = pl.multiple_of(
                    lax.shift_right_logical(i, 7) * 128, 128)
                lanes_p.append(u & 127)
                lanes_q.append(i & 127)
                pcp.append(pltpu.async_copy(
                    pt_hbm.at[:, pl.ds(ublk, 128)],
                    pwin_v.at[bank].at[j], psem))
                qcp.append(pltpu.async_copy(
                    qt_hbm.at[:, pl.ds(iblk, 128)],
                    qwin_v.at[bank].at[j], qsem))
            pending.append((g, pcp, qcp, lanes_p, lanes_q))
            if len(pending) == _NBUF:
                extract(pending.pop(0))
        for st in pending:
            extract(st)
        pltpu.sync_copy(pcols_v, put_hbm.at[:, pl.ds(base, b_per_w)])
        pltpu.sync_copy(qcols_v, qit_hbm.at[:, pl.ds(base, b_per_w)])

    return gather(PT, QT, user_id, item_id)


def _matmul_tc(PuT, QiT, tm=1024):
    def body(pt_ref, qt_ref, o_ref):
        o_ref[...] = lax.dot_general(
            pt_ref[...], qt_ref[...],
            dimension_numbers=(((0,), (0,)), ((), ())),
            preferred_element_type=jnp.float32,
        )

    return pl.pallas_call(
        body,
        grid=(_B // tm,),
        in_specs=[
            pl.BlockSpec((_D, tm), lambda i: (0, i)),
            pl.BlockSpec((_D, _B), lambda i: (0, 0)),
        ],
        out_specs=pl.BlockSpec((tm, _B), lambda i: (i, 0)),
        out_shape=jax.ShapeDtypeStruct((_B, _B), jnp.float32),
        compiler_params=pltpu.CompilerParams(
            fuse_transposed_lhs_in_matmul=True),
    )(PuT, QiT)


def kernel(user_id, item_id, P, Q):
    PT = P.T
    QT = Q.T
    PuT, QiT = _gather_sc(PT, QT, user_id, item_id)
    return _matmul_tc(PuT, QiT)


# speedup vs baseline: 1.0235x; 1.0235x over previous
import functools

import jax
import jax.numpy as jnp
from jax import lax
from jax.experimental import pallas as pl
from jax.experimental.pallas import tpu as pltpu
from jax.experimental.pallas import tpu_sc as plsc

_B = 4096
_D = 16
_GRP = 8
_NBUF = 3


def _gather_sc(PT, QT, user_id, item_id):
    info = plsc.get_sparse_core_info()
    nc, ns = info.num_cores, info.num_subcores
    nw = nc * ns
    b_per_w = _B // nw
    n_grp = b_per_w // _GRP

    mesh = plsc.VectorSubcoreMesh(core_axis_name="c", subcore_axis_name="s")

    @functools.partial(
        pl.kernel,
        mesh=mesh,
        out_type=[
            jax.ShapeDtypeStruct((_D, _B), jnp.float32),
            jax.ShapeDtypeStruct((_D, _B), jnp.float32),
        ],
        scratch_types=[
            pltpu.VMEM((b_per_w,), jnp.int32),
            pltpu.VMEM((b_per_w,), jnp.int32),
            pltpu.VMEM((_NBUF, _GRP, _D, 128), jnp.float32),
            pltpu.VMEM((_NBUF, _GRP, _D, 128), jnp.float32),
            pltpu.VMEM((_D, b_per_w), jnp.float32),
            pltpu.VMEM((_D, b_per_w), jnp.float32),
            pltpu.SemaphoreType.DMA,
            pltpu.SemaphoreType.DMA,
        ],
        compiler_params=pltpu.CompilerParams(needs_layout_passes=False),
    )
    def gather(pt_hbm, qt_hbm, uid_hbm, iid_hbm, put_hbm, qit_hbm,
               uidx_v, iidx_v, pwin_v, qwin_v, pcols_v, qcols_v, psem, qsem):
        wid = lax.axis_index("s") * nc + lax.axis_index("c")
        base = wid * b_per_w
        pltpu.sync_copy(uid_hbm.at[pl.ds(base, b_per_w)], uidx_v)
        pltpu.sync_copy(iid_hbm.at[pl.ds(base, b_per_w)], iidx_v)
        rows16 = jnp.arange(_D, dtype=jnp.int32)

        def extract(state):
            g, pcp, qcp, lanes_p, lanes_q = state
            bank = g % _NBUF
            for j in range(_GRP):
                col = _GRP * g + j
                colvec = jnp.full((_D,), col, dtype=jnp.int32)
                pcp[j].wait()
                pv = plsc.load_gather(
                    pwin_v.at[bank].at[j],
                    [rows16, jnp.full((_D,), lanes_p[j], dtype=jnp.int32)])
                plsc.store_scatter(pcols_v, [rows16, colvec], pv)
                qcp[j].wait()
                qv = plsc.load_gather(
                    qwin_v.at[bank].at[j],
                    [rows16, jnp.full((_D,), lanes_q[j], dtype=jnp.int32)])
                plsc.store_scatter(qcols_v, [rows16, colvec], qv)

        pending = []
        uvec = ivec = None
        for g in range(n_grp):
            if g % 2 == 0:
                uvec = uidx_v[pl.ds(16 * (g // 2), 16)]
                ivec = iidx_v[pl.ds(16 * (g // 2), 16)]
            off = _GRP * (g % 2)
            bank = g % _NBUF
            pcp, qcp, lanes_p, lanes_q = [], [], [], []
            for j in range(_GRP):
                u = uvec[off + j]
                i = ivec[off + j]
                ublk = pl.multiple_of(
                    lax.shift_right_logical(u, 7) * 128, 128)
                iblk = pl.multiple_of(
                    lax.shift_right_logical(i, 7) * 128, 128)
                lanes_p.append(u & 127)
                lanes_q.append(i & 127)
                pcp.append(pltpu.async_copy(
                    pt_hbm.at[:, pl.ds(ublk, 128)],
                    pwin_v.at[bank].at[j], psem))
                qcp.append(pltpu.async_copy(
                    qt_hbm.at[:, pl.ds(iblk, 128)],
                    qwin_v.at[bank].at[j], qsem))
            pending.append((g, pcp, qcp, lanes_p, lanes_q))
            if len(pending) == _NBUF:
                extract(pending.pop(0))
        for st in pending:
            extract(st)
        pltpu.sync_copy(pcols_v, put_hbm.at[:, pl.ds(base, b_per_w)])
        pltpu.sync_copy(qcols_v, qit_hbm.at[:, pl.ds(base, b_per_w)])

    return gather(PT, QT, user_id, item_id)


def _matmul_tc(PuT, QiT, tm=512):
    def body(pt_ref, qt_ref, o_ref):
        o_ref[...] = lax.dot_general(
            pt_ref[...], qt_ref[...],
            dimension_numbers=(((0,), (0,)), ((), ())),
            preferred_element_type=jnp.float32,
        )

    return pl.pallas_call(
        body,
        grid=(_B // tm,),
        in_specs=[
            pl.BlockSpec((_D, tm), lambda i: (0, i)),
            pl.BlockSpec((_D, _B), lambda i: (0, 0)),
        ],
        out_specs=pl.BlockSpec((tm, _B), lambda i: (i, 0)),
        out_shape=jax.ShapeDtypeStruct((_B, _B), jnp.float32),
        compiler_params=pltpu.CompilerParams(
            fuse_transposed_lhs_in_matmul=True),
    )(PuT, QiT)


def kernel(user_id, item_id, P, Q):
    PT = P.T
    QT = Q.T
    PuT, QiT = _gather_sc(PT, QT, user_id, item_id)
    return _matmul_tc(PuT, QiT)
